# baseline (speedup 1.0000x reference)
import functools

import jax
import jax.numpy as jnp
from jax import lax
from jax.experimental import pallas as pl
from jax.experimental.pallas import tpu as pltpu
from jax.experimental.pallas import tpu_sc as plsc

_N = 10000
_E = 320000
_D = 128
_NW = 32
_NPW = 320
_NPAD = _NW * _NPW
_C = 2000
_NCHUNK = _E // _C
_SUB = _C // 16
_K = 128


def _drain(h_hbm, sbuf, abuf_s, abuf_d, rows, acc, sem):
    for t in range(_K // 16):
        sbuf[pl.ds(t * 16, 16)] = abuf_s[pl.ds(t * 16, 16)]
    pltpu.async_copy(h_hbm.at[sbuf], rows, sem).wait()

    def row_body(r, carry):
        d = abuf_d[pl.ds(r, 16)][0]
        for c in range(_D // 16):
            sl = pl.ds(c * 16, 16)
            acc[d, sl] = jnp.maximum(acc[d, sl], rows[r, sl])
        return carry

    lax.fori_loop(0, _K, row_body, 0)


def _sc_body(h_hbm, src_hbm, dst_hbm, out_hbm,
             acc, schunk, dchunk, abuf_s, abuf_d, sbuf, rows, cbuf, mbuf, sem):
    nc = 2
    wid = lax.axis_index("s") * nc + lax.axis_index("c")
    lo = wid * _NPW

    neg = jnp.full((16,), -jnp.inf, dtype=jnp.float32)

    def init_body(r, carry):
        for c in range(_D // 16):
            acc[r, pl.ds(c * 16, 16)] = neg
        return carry

    lax.fori_loop(0, _NPW, init_body, 0)

    zero16 = jnp.zeros((16,), dtype=jnp.int32)
    trash16 = jnp.full((16,), _NPW, dtype=jnp.int32)
    for i in range(_K // 16 + 1):
        abuf_s[pl.ds(i * 16, 16)] = zero16
        abuf_d[pl.ds(i * 16, 16)] = trash16
    for i in range(3):
        cbuf[pl.ds(i * 16, 16)] = zero16
    for i in range(2):
        mbuf[pl.ds(i * 16, 16)] = zero16

    iota16 = lax.iota(jnp.int32, 16)

    def chunk_body(ch, cnt):
        pltpu.sync_copy(src_hbm.at[pl.ds(ch * _C, _C)], schunk.at[pl.ds(0, _C)])
        pltpu.sync_copy(dst_hbm.at[pl.ds(ch * _C, _C)], dchunk.at[pl.ds(0, _C)])

        def sub_body(i, cnt):
            dvec = dchunk[pl.ds(i * 16, 16)]
            dloc = dvec - lo
            mask = (dloc >= 0) & (dloc < _NPW)
            m = jnp.where(mask, 1, 0)
            mbuf[pl.ds(0, 16)] = m
            v = m + (m * iota16 << 5)
            for sh in (1, 2, 4, 8):
                cbuf[pl.ds(16, 16)] = v
                v = v + cbuf[pl.ds(16 - sh, 16)]
            cbuf[pl.ds(16, 16)] = v
            tot = cbuf[pl.ds(16, 16)][15]
            c = tot & 31

            @pl.when(c == 1)
            def _():
                lane = tot >> 5
                svk = schunk[pl.ds(i * 16 + lane, 16)][0]
                dlk = dchunk[pl.ds(i * 16 + lane, 16)][0] - lo
                abuf_s[pl.ds(cnt, 16)] = jnp.full((16,), svk, jnp.int32)
                abuf_d[pl.ds(cnt, 16)] = jnp.full((16,), dlk, jnp.int32)

            @pl.when(c >= 2)
            def _():
                def lane_body(k, cnt2):
                    mk = mbuf[pl.ds(k, 16)][0]

                    @pl.when(mk == 1)
                    def _():
                        svk = schunk[pl.ds(i * 16 + k, 16)][0]
                        dlk = dchunk[pl.ds(i * 16 + k, 16)][0] - lo
                        abuf_s[pl.ds(cnt2, 16)] = jnp.full((16,), svk, jnp.int32)
                        abuf_d[pl.ds(cnt2, 16)] = jnp.full((16,), dlk, jnp.int32)

                    return cnt2 + mk

                lax.fori_loop(0, 16, lane_body, cnt)

            cnt = cnt + c

            @pl.when(cnt >= _K - 16)
            def _():
                _drain(h_hbm, sbuf, abuf_s, abuf_d, rows, acc, sem)

            return jnp.where(cnt >= _K - 16, 0, cnt)

        return lax.fori_loop(0, _SUB, sub_body, cnt)

    cnt = lax.fori_loop(0, _NCHUNK, chunk_body, jnp.int32(0))

    _drain(h_hbm, sbuf, abuf_s, abuf_d, rows, acc, sem)
    pltpu.sync_copy(acc.at[pl.ds(0, _NPW)], out_hbm.at[pl.ds(lo, _NPW)])


_sc_gather_max = functools.partial(
    pl.kernel,
    out_type=jax.ShapeDtypeStruct((_NPAD, _D), jnp.float32),
    mesh=plsc.VectorSubcoreMesh(core_axis_name="c", subcore_axis_name="s"),
    scratch_types=[
        pltpu.VMEM((_NPW + 1, _D), jnp.float32),
        pltpu.VMEM((_C + 16,), jnp.int32),
        pltpu.VMEM((_C + 16,), jnp.int32),
        pltpu.VMEM((_K + 16,), jnp.int32),
        pltpu.VMEM((_K + 16,), jnp.int32),
        pltpu.VMEM((_K,), jnp.int32),
        pltpu.VMEM((_K, _D), jnp.float32),
        pltpu.VMEM((48,), jnp.int32),
        pltpu.VMEM((32,), jnp.int32),
        pltpu.SemaphoreType.DMA,
    ],
)(_sc_body)


def _tc_body(h_ref, a_ref, wt_ref, b_ref, o_ref, *, relu):
    a = a_ref[...]
    x = h_ref[...] + jnp.where(a == -jnp.inf, 0.0, a)
    y = jnp.dot(x, wt_ref[...], preferred_element_type=jnp.float32) + b_ref[...]
    if relu:
        y = jnp.maximum(y, 0.0)
    o_ref[...] = y


def _tc_linear(h, agg, wt, b, relu):
    blk = 1000
    return pl.pallas_call(
        functools.partial(_tc_body, relu=relu),
        grid=(_N // blk,),
        in_specs=[
            pl.BlockSpec((blk, _D), lambda i: (i, 0)),
            pl.BlockSpec((blk, _D), lambda i: (i, 0)),
            pl.BlockSpec((_D, _D), lambda i: (0, 0)),
            pl.BlockSpec((1, _D), lambda i: (0, 0)),
        ],
        out_specs=pl.BlockSpec((blk, _D), lambda i: (i, 0)),
        out_shape=jax.ShapeDtypeStruct((_N, _D), jnp.float32),
    )(h, agg, wt, b.reshape(1, _D))


def kernel(h, edge_index, W1, b1, W2, b2):
    src = edge_index[0]
    dst = edge_index[1]
    agg1 = _sc_gather_max(h, src, dst)
    h1 = _tc_linear(h, agg1[:_N], W1.T, b1, relu=True)
    agg2 = _sc_gather_max(h1, src, dst)
    return _tc_linear(h1, agg2[:_N], W2.T, b2, relu=False)


# speedup vs baseline: 2.5407x; 2.5407x over previous
import functools

import jax
import jax.numpy as jnp
from jax import lax
from jax.experimental import pallas as pl
from jax.experimental.pallas import tpu as pltpu
from jax.experimental.pallas import tpu_sc as plsc

_N = 10000
_E = 320000
_D = 128
_NW = 32
_NPW = 320
_NPAD = _NW * _NPW
_EPW = _E // _NW
_C = 2000
_NCHUNK = _EPW // _C
_K = 128
_BSTRIDE = _K + 16
_NBLK = _EPW // _K + 1
_RCAP = _NBLK * _K
_CSTRIDE = 16
_DIVM = 13108
_DIVS = 22


def _tile_id():
    return lax.axis_index("s") * 2 + lax.axis_index("c")


def _sc_bin_body(src_hbm, dst_hbm, sidx_hbm, didx_hbm, cnts_hbm,
                 schunk, dchunk, bs, bd, cntb, nflb, cbuf, sem):
    wid = _tile_id()

    zero16 = jnp.zeros((16,), dtype=jnp.int32)
    trash16 = jnp.full((16,), _NPW, dtype=jnp.int32)
    for o in range(_NW):
        cntb[pl.ds(o * _CSTRIDE, 16)] = zero16
        nflb[pl.ds(o * _CSTRIDE, 16)] = zero16
        for t in range(_BSTRIDE // 16):
            bs[pl.ds(o * _BSTRIDE + t * 16, 16)] = zero16
            bd[pl.ds(o * _BSTRIDE + t * 16, 16)] = trash16

    def flush(o, nf, valid):
        base = (wid * _NW) * _RCAP + o * _RCAP + nf * _K
        pltpu.sync_copy(bs.at[pl.ds(o * _BSTRIDE, _K)],
                        sidx_hbm.at[pl.ds(base, _K)])
        pltpu.sync_copy(bd.at[pl.ds(o * _BSTRIDE, _K)],
                        didx_hbm.at[pl.ds(base, _K)])
        del valid

    def chunk_body(ch, carry):
        pltpu.sync_copy(src_hbm.at[pl.ds(wid * _EPW + ch * _C, _C)],
                        schunk.at[pl.ds(0, _C)])
        pltpu.sync_copy(dst_hbm.at[pl.ds(wid * _EPW + ch * _C, _C)],
                        dchunk.at[pl.ds(0, _C)])

        def edge_body(e, carry2):
            sv = schunk[pl.ds(e, 16)][0]
            dv = dchunk[pl.ds(e, 16)][0]
            o = (dv * _DIVM) >> _DIVS
            dl = dv - o * _NPW
            co = cntb[pl.ds(o * _CSTRIDE, 16)][0]
            bs[pl.ds(o * _BSTRIDE + co, 16)] = jnp.full((16,), sv, jnp.int32)
            bd[pl.ds(o * _BSTRIDE + co, 16)] = jnp.full((16,), dl, jnp.int32)

            @pl.when(co == _K - 1)
            def _():
                nf = nflb[pl.ds(o * _CSTRIDE, 16)][0]
                flush(o, nf, _K)
                nflb[pl.ds(o * _CSTRIDE, 16)] = jnp.full((16,), nf + 1, jnp.int32)

            cntb[pl.ds(o * _CSTRIDE, 16)] = jnp.full(
                (16,), jnp.where(co == _K - 1, 0, co + 1), jnp.int32)
            return carry2

        lax.fori_loop(0, _C, edge_body, 0)
        return carry

    lax.fori_loop(0, _NCHUNK, chunk_body, 0)

    def tail_body(o, carry):
        nf = nflb[pl.ds(o * _CSTRIDE, 16)][0]
        flush(o, nf, 0)
        nflb[pl.ds(o * _CSTRIDE, 16)] = jnp.full((16,), nf + 1, jnp.int32)
        return carry

    lax.fori_loop(0, _NW, tail_body, 0)

    pltpu.sync_copy(nflb, cnts_hbm.at[pl.ds(wid * _NW * _CSTRIDE, _NW * _CSTRIDE)])


_sc_bin = functools.partial(
    pl.kernel,
    out_type=(
        jax.ShapeDtypeStruct((_NW * _NW * _RCAP,), jnp.int32),
        jax.ShapeDtypeStruct((_NW * _NW * _RCAP,), jnp.int32),
        jax.ShapeDtypeStruct((_NW * _NW * _CSTRIDE,), jnp.int32),
    ),
    mesh=plsc.VectorSubcoreMesh(core_axis_name="c", subcore_axis_name="s"),
    scratch_types=[
        pltpu.VMEM((_C + 16,), jnp.int32),
        pltpu.VMEM((_C + 16,), jnp.int32),
        pltpu.VMEM((_NW * _BSTRIDE,), jnp.int32),
        pltpu.VMEM((_NW * _BSTRIDE,), jnp.int32),
        pltpu.VMEM((_NW * _CSTRIDE,), jnp.int32),
        pltpu.VMEM((_NW * _CSTRIDE,), jnp.int32),
        pltpu.VMEM((32,), jnp.int32),
        pltpu.SemaphoreType.DMA,
    ],
)(_sc_bin_body)


def _sc_drain_body(h_hbm, sidx_hbm, didx_hbm, cnts_hbm, out_hbm,
                   acc, cvm, sbuf, dbat, rows, sem):
    wid = _tile_id()
    lo = wid * _NPW

    neg = jnp.full((16,), -jnp.inf, dtype=jnp.float32)

    def init_body(r, carry):
        for c in range(_D // 16):
            acc[r, pl.ds(c * 16, 16)] = neg
        return carry

    lax.fori_loop(0, _NPW, init_body, 0)

    pltpu.sync_copy(cnts_hbm, cvm)

    def src_body(t, carry):
        nfl = cvm[pl.ds(t * _NW * _CSTRIDE + wid * _CSTRIDE, 16)][0]
        rbase = (t * _NW + wid) * _RCAP

        def blk_body(b, carry2):
            base = rbase + b * _K
            pltpu.sync_copy(sidx_hbm.at[pl.ds(base, _K)], sbuf)
            pltpu.sync_copy(didx_hbm.at[pl.ds(base, _K)], dbat.at[pl.ds(0, _K)])
            pltpu.async_copy(h_hbm.at[sbuf], rows, sem).wait()

            def row_body(r, carry3):
                d = dbat[pl.ds(r, 16)][0]
                for c in range(_D // 16):
                    sl = pl.ds(c * 16, 16)
                    acc[d, sl] = jnp.maximum(acc[d, sl], rows[r, sl])
                return carry3

            lax.fori_loop(0, _K, row_body, 0)
            return carry2

        lax.fori_loop(0, nfl, blk_body, 0)
        return carry

    lax.fori_loop(0, _NW, src_body, 0)

    pltpu.sync_copy(acc.at[pl.ds(0, _NPW)], out_hbm.at[pl.ds(lo, _NPW)])


_sc_drain = functools.partial(
    pl.kernel,
    out_type=jax.ShapeDtypeStruct((_NPAD, _D), jnp.float32),
    mesh=plsc.VectorSubcoreMesh(core_axis_name="c", subcore_axis_name="s"),
    scratch_types=[
        pltpu.VMEM((_NPW + 1, _D), jnp.float32),
        pltpu.VMEM((_NW * _NW * _CSTRIDE,), jnp.int32),
        pltpu.VMEM((_K,), jnp.int32),
        pltpu.VMEM((_K + 16,), jnp.int32),
        pltpu.VMEM((_K, _D), jnp.float32),
        pltpu.SemaphoreType.DMA,
    ],
)(_sc_drain_body)


def _tc_body(h_ref, a_ref, wt_ref, b_ref, o_ref, *, relu):
    a = a_ref[...]
    x = h_ref[...] + jnp.where(a == -jnp.inf, 0.0, a)
    y = jnp.dot(x, wt_ref[...], preferred_element_type=jnp.float32) + b_ref[...]
    if relu:
        y = jnp.maximum(y, 0.0)
    o_ref[...] = y


def _tc_linear(h, agg, wt, b, relu):
    blk = 1000
    return pl.pallas_call(
        functools.partial(_tc_body, relu=relu),
        grid=(_N // blk,),
        in_specs=[
            pl.BlockSpec((blk, _D), lambda i: (i, 0)),
            pl.BlockSpec((blk, _D), lambda i: (i, 0)),
            pl.BlockSpec((_D, _D), lambda i: (0, 0)),
            pl.BlockSpec((1, _D), lambda i: (0, 0)),
        ],
        out_specs=pl.BlockSpec((blk, _D), lambda i: (i, 0)),
        out_shape=jax.ShapeDtypeStruct((_N, _D), jnp.float32),
    )(h, agg, wt, b.reshape(1, _D))


def kernel(h, edge_index, W1, b1, W2, b2):
    src = edge_index[0]
    dst = edge_index[1]
    sidx, didx, cnts = _sc_bin(src, dst)
    agg1 = _sc_drain(h, sidx, didx, cnts)
    h1 = _tc_linear(h, agg1[:_N], W1.T, b1, relu=True)
    agg2 = _sc_drain(h1, sidx, didx, cnts)
    return _tc_linear(h1, agg2[:_N], W2.T, b2, relu=False)
